# baseline (speedup 1.0000x reference)
import functools

import jax
import jax.numpy as jnp
from jax import lax
from jax.experimental import pallas as pl
from jax.experimental.pallas import tpu as pltpu
from jax.experimental.pallas import tpu_sc as plsc

N = 50000
E = 800000
H = 64
HH = 32
D_IN = 9

NC = 2
NS = 16
CHUNK = 128

NPAD = 50176
RPT = NPAD // NS
NCHUNK = 408
EPT = NCHUNK * CHUNK
EPAD = NS * EPT
NBLK = 8
NG = NCHUNK // NBLK
NB = 4
GAP = 2

GB = NPAD // (NC * NS)
GCHUNK = 112
GN = GB // GCHUNK

_sc_mesh = plsc.VectorSubcoreMesh(core_axis_name="c", subcore_axis_name="s")
_sc_params = pltpu.CompilerParams(use_tc_tiling_on_sc=False)


@functools.partial(
    pl.kernel,
    out_type=jax.ShapeDtypeStruct((NPAD, H), jnp.float32),
    mesh=_sc_mesh,
    compiler_params=_sc_params,
    scratch_types=[
        pltpu.VMEM((GN, 1, GCHUNK), jnp.int32),
        pltpu.VMEM((NB, GCHUNK, H), jnp.float32),
        pltpu.SemaphoreType.DMA((NB,)),
        pltpu.SemaphoreType.DMA((NB,)),
    ],
)
def _sc_embed(table, idx, out, idx_v, rows_v, gsem, osem):
    c = lax.axis_index("c")
    s = lax.axis_index("s")
    wid = s * NC + c
    base = wid * GB
    pltpu.sync_copy(idx.at[pl.ds(wid * GN, GN)], idx_v)
    for b in range(GAP):
        pltpu.async_copy(table.at[idx_v.at[b, 0]], rows_v.at[b],
                         gsem.at[b])
    for j in range(GN):
        b = j % NB
        pltpu.make_async_copy(table.at[pl.ds(0, GCHUNK)], rows_v.at[b],
                              gsem.at[b]).wait()
        pltpu.async_copy(rows_v.at[b],
                         out.at[pl.ds(base + j * GCHUNK, GCHUNK)],
                         osem.at[b])
        jn = j + GAP
        if jn < GN:
            bg = jn % NB
            if jn >= NB:
                pltpu.make_async_copy(table.at[pl.ds(0, GCHUNK)],
                                      rows_v.at[bg], osem.at[bg]).wait()
            pltpu.async_copy(table.at[idx_v.at[jn, 0]], rows_v.at[bg],
                             gsem.at[bg])
    for j in range(GN - min(GN, NB), GN):
        b = j % NB
        pltpu.make_async_copy(table.at[pl.ds(0, GCHUNK)], rows_v.at[b],
                              osem.at[b]).wait()


def _make_segsum(with_counts):
    out_types = [jax.ShapeDtypeStruct((2, NPAD, HH), jnp.float32)]
    scratch = [
        pltpu.VMEM((2, NBLK, 2, 1, CHUNK), jnp.int32),
        pltpu.VMEM((NB, CHUNK, HH), jnp.float32),
        pltpu.VMEM_SHARED((NPAD, HH), jnp.float32),
        pltpu.SemaphoreType.DMA((2,)),
        pltpu.SemaphoreType.DMA((NB,)),
        pltpu.SemaphoreType.DMA((NB,)),
    ]
    if with_counts:
        out_types.append(jax.ShapeDtypeStruct((2 * NPAD,), jnp.float32))
        scratch += [
            pltpu.VMEM((CHUNK,), jnp.float32),
            pltpu.VMEM_SHARED((NPAD,), jnp.float32),
            pltpu.SemaphoreType.DMA((NB,)),
        ]

    def body(y2, em, zeros2, zeros1, *refs):
        if with_counts:
            (out, outc, ibuf, rows, acc, isem, gsem, ssem,
             ones_v, cacc, csem) = refs
        else:
            out, ibuf, rows, acc, isem, gsem, ssem = refs
        c = lax.axis_index("c")
        s = lax.axis_index("s")
        r0 = s * RPT
        yc = y2.at[c]
        gbase = s * NG

        def wait_rows(sem):
            pltpu.make_async_copy(zeros2.at[pl.ds(0, CHUNK)],
                                  rows.at[0], sem).wait()

        def wait_ones(sem):
            pltpu.make_async_copy(zeros1.at[pl.ds(0, CHUNK)], ones_v,
                                  sem).wait()

        pltpu.sync_copy(zeros2.at[pl.ds(r0, RPT)], acc.at[pl.ds(r0, RPT)])
        if with_counts:
            pltpu.sync_copy(zeros1.at[pl.ds(r0, RPT)],
                            cacc.at[pl.ds(r0, RPT)])
            for i in range(CHUNK // 16):
                ones_v[pl.ds(i * 16, 16)] = jnp.ones((16,), jnp.float32)
        pltpu.sync_copy(em.at[gbase], ibuf.at[0])
        pltpu.async_copy(em.at[gbase + 1], ibuf.at[1], isem.at[1])
        for b in range(GAP):
            pltpu.async_copy(yc.at[ibuf.at[0, b, 0, 0]], rows.at[b],
                             gsem.at[b])
        plsc.subcore_barrier()

        def group(g, carry):
            p = lax.rem(g, 2)
            for k in range(NBLK):
                b = k % NB
                if k == 2:
                    @pl.when(jnp.logical_and(g >= 1, g + 1 < NG))
                    def _():
                        pltpu.async_copy(em.at[gbase + g + 1],
                                         ibuf.at[1 - p], isem.at[1 - p])
                if k == NBLK - GAP:
                    @pl.when(g + 1 < NG)
                    def _():
                        pltpu.make_async_copy(
                            em.at[gbase], ibuf.at[1 - p],
                            isem.at[1 - p]).wait()
                wait_rows(gsem.at[b])
                pltpu.async_copy(rows.at[b], acc.at[ibuf.at[p, k, 1, 0]],
                                 ssem.at[b], add=True)
                if with_counts:
                    pltpu.async_copy(ones_v,
                                     cacc.at[ibuf.at[p, k, 1, 0]],
                                     csem.at[b], add=True)
                bg = (k + GAP) % NB

                def drain():
                    wait_rows(ssem.at[bg])
                    if with_counts:
                        wait_ones(csem.at[bg])

                if k < GAP:
                    @pl.when(g > 0)
                    def _():
                        drain()
                else:
                    drain()
                kn = k + GAP
                if kn < NBLK:
                    pltpu.async_copy(yc.at[ibuf.at[p, kn, 0, 0]],
                                     rows.at[bg], gsem.at[bg])
                else:
                    @pl.when(g + 1 < NG)
                    def _():
                        pltpu.async_copy(
                            yc.at[ibuf.at[1 - p, kn - NBLK, 0, 0]],
                            rows.at[bg], gsem.at[bg])
            return carry

        lax.fori_loop(0, NG, group, 0)
        for k in range(NBLK - GAP, NBLK):
            wait_rows(ssem.at[k % NB])
            if with_counts:
                wait_ones(csem.at[k % NB])
        plsc.subcore_barrier()
        pltpu.sync_copy(acc.at[pl.ds(r0, RPT)], out.at[c, pl.ds(r0, RPT)])
        if with_counts:
            pltpu.sync_copy(cacc.at[pl.ds(r0, RPT)],
                            outc.at[pl.ds(c * NPAD + r0, RPT)])

    return pl.kernel(
        body,
        out_type=tuple(out_types) if with_counts else out_types[0],
        mesh=_sc_mesh,
        compiler_params=_sc_params,
        scratch_types=scratch,
    )


_sc_segsum = _make_segsum(False)
_sc_segsum_cnt = _make_segsum(True)


_BM = 512


def _linear_body(x_ref, w_ref, b_ref, o_ref):
    o_ref[...] = x_ref[...] @ w_ref[...] + b_ref[...]


def _tc_linear(x, w, b):
    m, k = x.shape
    h = w.shape[1]
    return pl.pallas_call(
        _linear_body,
        grid=(m // _BM,),
        in_specs=[
            pl.BlockSpec((_BM, k), lambda i: (i, 0)),
            pl.BlockSpec((k, h), lambda i: (0, 0)),
            pl.BlockSpec((1, h), lambda i: (0, 0)),
        ],
        out_specs=pl.BlockSpec((_BM, h), lambda i: (i, 0)),
        out_shape=jax.ShapeDtypeStruct((m, h), jnp.float32),
    )(x, w, b)


def _proj_body(x_ref, w_ref, o_ref):
    o_ref[0] = x_ref[...] @ w_ref[0]


def _tc_proj(x, w):
    w2 = w.reshape(H, 2, HH).transpose(1, 0, 2)
    return pl.pallas_call(
        _proj_body,
        grid=(NPAD // _BM, 2),
        in_specs=[
            pl.BlockSpec((_BM, H), lambda i, h: (i, 0)),
            pl.BlockSpec((1, H, HH), lambda i, h: (h, 0, 0)),
        ],
        out_specs=pl.BlockSpec((1, _BM, HH), lambda i, h: (h, i, 0)),
        out_shape=jax.ShapeDtypeStruct((2, NPAD, HH), jnp.float32),
    )(x, w2)


def _tail_body(s0_ref, s1_ref, cnt_ref, b_ref, x_ref, w_ref, o_ref):
    agg = jnp.concatenate([s0_ref[0], s1_ref[0]], axis=1)
    inv = 1.0 / jnp.maximum(cnt_ref[...], 1.0)
    o_ref[...] = jnp.maximum(
        agg * inv + b_ref[...] + x_ref[...] @ w_ref[...], 0.0)


def _tc_tail(s2, cnt2d, b, x, wr):
    return pl.pallas_call(
        _tail_body,
        grid=(NPAD // _BM,),
        in_specs=[
            pl.BlockSpec((1, _BM, HH), lambda i: (0, i, 0)),
            pl.BlockSpec((1, _BM, HH), lambda i: (1, i, 0)),
            pl.BlockSpec((_BM, 1), lambda i: (i, 0)),
            pl.BlockSpec((1, H), lambda i: (0, 0)),
            pl.BlockSpec((_BM, H), lambda i: (i, 0)),
            pl.BlockSpec((H, H), lambda i: (0, 0)),
        ],
        out_specs=pl.BlockSpec((_BM, H), lambda i: (i, 0)),
        out_shape=jax.ShapeDtypeStruct((NPAD, H), jnp.float32),
    )(s2, s2, cnt2d, b, x, wr)


def kernel(x_user, x_recipe, edge_u2r, edge_r2u, emb_user, W_in, b_in,
           W_ur0, Wr_ur0, b_ur0, W_ru0, Wr_ru0, b_ru0,
           W_ur1, Wr_ur1, b_ur1, W_ru1, Wr_ru1, b_ru1):
    f32 = jnp.float32

    idx_u = jnp.pad(x_user.astype(jnp.int32),
                    (0, NPAD - N)).reshape(-1, 1, GCHUNK)
    xr = jnp.pad(x_recipe, ((0, NPAD - N), (0, 16 - D_IN)))
    w_in16 = jnp.pad(W_in, ((0, 16 - D_IN), (0, 0)))

    def prep_edges(edge):
        src = jnp.pad(edge[0].astype(jnp.int32), (0, EPAD - E))
        dst = jnp.pad(edge[1].astype(jnp.int32), (0, EPAD - E),
                      constant_values=N)
        return jnp.stack([src.reshape(NS * NG, NBLK, 1, CHUNK),
                          dst.reshape(NS * NG, NBLK, 1, CHUNK)], axis=2)

    em_u2r = prep_edges(edge_u2r)
    em_r2u = prep_edges(edge_r2u)

    zeros1 = jnp.zeros((NPAD,), f32)
    zeros2 = jnp.zeros((NPAD, HH), f32)

    b2 = {k: v.reshape(1, H) for k, v in dict(
        b_in=b_in, b_ur0=b_ur0, b_ru0=b_ru0, b_ur1=b_ur1, b_ru1=b_ru1).items()}

    h_u = _sc_embed(emb_user, idx_u)
    h_r = _tc_linear(xr, w_in16, b2["b_in"])

    y_u = _tc_proj(h_u, W_ur0)
    y_r = _tc_proj(h_r, W_ru0)
    s_r, cnts_r = _sc_segsum_cnt(y_u, em_u2r, zeros2, zeros1)
    s_u, cnts_u = _sc_segsum_cnt(y_r, em_r2u, zeros2, zeros1)
    cnt_r = cnts_r[:NPAD].reshape(NPAD, 1)
    cnt_u = cnts_u[:NPAD].reshape(NPAD, 1)
    h_r1 = _tc_tail(s_r, cnt_r, b2["b_ur0"], h_r, Wr_ur0)
    h_u1 = _tc_tail(s_u, cnt_u, b2["b_ru0"], h_u, Wr_ru0)

    y_u = _tc_proj(h_u1, W_ur1)
    y_r = _tc_proj(h_r1, W_ru1)
    s_r = _sc_segsum(y_u, em_u2r, zeros2, zeros1)
    s_u = _sc_segsum(y_r, em_r2u, zeros2, zeros1)
    out_r = _tc_tail(s_r, cnt_r, b2["b_ur1"], h_r1, Wr_ur1)
    out_u = _tc_tail(s_u, cnt_u, b2["b_ru1"], h_u1, Wr_ru1)

    return out_u[:N], out_r[:N]


# speedup vs baseline: 1.1839x; 1.1839x over previous
import functools

import jax
import jax.numpy as jnp
from jax import lax
from jax.experimental import pallas as pl
from jax.experimental.pallas import tpu as pltpu
from jax.experimental.pallas import tpu_sc as plsc

N = 50000
E = 800000
H = 64
HH = 32
D_IN = 9

NC = 2
NS = 16
CHUNK = 128

NPAD = 50176
RPT = NPAD // NS
NCHUNK = 408
EPT = NCHUNK * CHUNK
EPAD = NS * EPT
NBLK = 8
NG = NCHUNK // NBLK
NB = 4
GAP = 2

GB = NPAD // (NC * NS)
GCHUNK = 112
GN = GB // GCHUNK

_sc_mesh = plsc.VectorSubcoreMesh(core_axis_name="c", subcore_axis_name="s")
_sc_params = pltpu.CompilerParams(use_tc_tiling_on_sc=False)


@functools.partial(
    pl.kernel,
    out_type=jax.ShapeDtypeStruct((NPAD, H), jnp.float32),
    mesh=_sc_mesh,
    compiler_params=_sc_params,
    scratch_types=[
        pltpu.VMEM((GN, 1, GCHUNK), jnp.int32),
        pltpu.VMEM((NB, GCHUNK, H), jnp.float32),
        pltpu.SemaphoreType.DMA((NB,)),
        pltpu.SemaphoreType.DMA((NB,)),
    ],
)
def _sc_embed(table, idx, out, idx_v, rows_v, gsem, osem):
    c = lax.axis_index("c")
    s = lax.axis_index("s")
    wid = s * NC + c
    base = wid * GB
    pltpu.sync_copy(idx.at[pl.ds(wid * GN, GN)], idx_v)
    for b in range(GAP):
        pltpu.async_copy(table.at[idx_v.at[b, 0]], rows_v.at[b],
                         gsem.at[b])
    for j in range(GN):
        b = j % NB
        pltpu.make_async_copy(table.at[pl.ds(0, GCHUNK)], rows_v.at[b],
                              gsem.at[b]).wait()
        pltpu.async_copy(rows_v.at[b],
                         out.at[pl.ds(base + j * GCHUNK, GCHUNK)],
                         osem.at[b])
        jn = j + GAP
        if jn < GN:
            bg = jn % NB
            if jn >= NB:
                pltpu.make_async_copy(table.at[pl.ds(0, GCHUNK)],
                                      rows_v.at[bg], osem.at[bg]).wait()
            pltpu.async_copy(table.at[idx_v.at[jn, 0]], rows_v.at[bg],
                             gsem.at[bg])
    for j in range(GN - min(GN, NB), GN):
        b = j % NB
        pltpu.make_async_copy(table.at[pl.ds(0, GCHUNK)], rows_v.at[b],
                              osem.at[b]).wait()


@functools.partial(
    pl.kernel,
    out_type=jax.ShapeDtypeStruct((2 * NPAD,), jnp.float32),
    mesh=_sc_mesh,
    compiler_params=_sc_params,
    scratch_types=[
        pltpu.VMEM((2, NBLK, 1, CHUNK), jnp.int32),
        pltpu.VMEM((CHUNK,), jnp.float32),
        pltpu.VMEM_SHARED((NPAD,), jnp.float32),
        pltpu.SemaphoreType.DMA((2,)),
        pltpu.SemaphoreType.DMA((NB,)),
    ],
)
def _sc_counts(dsts, zeros1, out, ibuf, ones_v, acc, isem, csem):
    c = lax.axis_index("c")
    s = lax.axis_index("s")
    r0 = s * RPT
    dc = dsts.at[c]
    gbase = s * NG

    def wait_ones(sem):
        pltpu.make_async_copy(zeros1.at[pl.ds(0, CHUNK)], ones_v,
                              sem).wait()

    pltpu.sync_copy(zeros1.at[pl.ds(r0, RPT)], acc.at[pl.ds(r0, RPT)])
    for i in range(CHUNK // 16):
        ones_v[pl.ds(i * 16, 16)] = jnp.ones((16,), jnp.float32)
    pltpu.sync_copy(dc.at[gbase], ibuf.at[0])
    pltpu.async_copy(dc.at[gbase + 1], ibuf.at[1], isem.at[1])
    plsc.subcore_barrier()

    def group(g, carry):
        p = lax.rem(g, 2)
        for k in range(NBLK):
            b = k % NB
            if k == 4:
                @pl.when(jnp.logical_and(g >= 1, g + 1 < NG))
                def _():
                    pltpu.async_copy(dc.at[gbase + g + 1], ibuf.at[1 - p],
                                     isem.at[1 - p])
            if k == NBLK - 1:
                @pl.when(g + 1 < NG)
                def _():
                    pltpu.make_async_copy(dc.at[gbase], ibuf.at[1 - p],
                                          isem.at[1 - p]).wait()
            if k < NB:
                @pl.when(g > 0)
                def _():
                    wait_ones(csem.at[b])
            else:
                wait_ones(csem.at[b])
            pltpu.async_copy(ones_v, acc.at[ibuf.at[p, k, 0]],
                             csem.at[b], add=True)
        return carry

    lax.fori_loop(0, NG, group, 0)
    for k in range(NBLK - NB, NBLK):
        wait_ones(csem.at[k % NB])
    plsc.subcore_barrier()
    pltpu.sync_copy(acc.at[pl.ds(r0, RPT)],
                    out.at[pl.ds(c * NPAD + r0, RPT)])


def _make_segsum():
    out_types = [jax.ShapeDtypeStruct((2, NPAD, HH), jnp.float32)]
    scratch = [
        pltpu.VMEM((2, NBLK, 2, 1, CHUNK), jnp.int32),
        pltpu.VMEM((NB, CHUNK, HH), jnp.float32),
        pltpu.VMEM_SHARED((NPAD, HH), jnp.float32),
        pltpu.SemaphoreType.DMA((2,)),
        pltpu.SemaphoreType.DMA((NB,)),
        pltpu.SemaphoreType.DMA((NB,)),
    ]
    with_counts = False

    def body(y2, em, zeros2, zeros1, *refs):
        if with_counts:
            (out, outc, ibuf, rows, acc, isem, gsem, ssem,
             ones_v, cacc, csem) = refs
        else:
            out, ibuf, rows, acc, isem, gsem, ssem = refs
        c = lax.axis_index("c")
        s = lax.axis_index("s")
        r0 = s * RPT
        yc = y2.at[c]
        gbase = s * NG

        def wait_rows(sem):
            pltpu.make_async_copy(zeros2.at[pl.ds(0, CHUNK)],
                                  rows.at[0], sem).wait()

        def wait_ones(sem):
            pltpu.make_async_copy(zeros1.at[pl.ds(0, CHUNK)], ones_v,
                                  sem).wait()

        pltpu.sync_copy(zeros2.at[pl.ds(r0, RPT)], acc.at[pl.ds(r0, RPT)])
        if with_counts:
            pltpu.sync_copy(zeros1.at[pl.ds(r0, RPT)],
                            cacc.at[pl.ds(r0, RPT)])
            for i in range(CHUNK // 16):
                ones_v[pl.ds(i * 16, 16)] = jnp.ones((16,), jnp.float32)
        pltpu.sync_copy(em.at[gbase], ibuf.at[0])
        pltpu.async_copy(em.at[gbase + 1], ibuf.at[1], isem.at[1])
        for b in range(GAP):
            pltpu.async_copy(yc.at[ibuf.at[0, b, 0, 0]], rows.at[b],
                             gsem.at[b])
        plsc.subcore_barrier()

        def group(g, carry):
            p = lax.rem(g, 2)
            for k in range(NBLK):
                b = k % NB
                if k == 2:
                    @pl.when(jnp.logical_and(g >= 1, g + 1 < NG))
                    def _():
                        pltpu.async_copy(em.at[gbase + g + 1],
                                         ibuf.at[1 - p], isem.at[1 - p])
                if k == NBLK - GAP:
                    @pl.when(g + 1 < NG)
                    def _():
                        pltpu.make_async_copy(
                            em.at[gbase], ibuf.at[1 - p],
                            isem.at[1 - p]).wait()
                wait_rows(gsem.at[b])
                pltpu.async_copy(rows.at[b], acc.at[ibuf.at[p, k, 1, 0]],
                                 ssem.at[b], add=True)
                if with_counts:
                    pltpu.async_copy(ones_v,
                                     cacc.at[ibuf.at[p, k, 1, 0]],
                                     csem.at[b], add=True)
                bg = (k + GAP) % NB

                def drain():
                    wait_rows(ssem.at[bg])
                    if with_counts:
                        wait_ones(csem.at[bg])

                if k < GAP:
                    @pl.when(g > 0)
                    def _():
                        drain()
                else:
                    drain()
                kn = k + GAP
                if kn < NBLK:
                    pltpu.async_copy(yc.at[ibuf.at[p, kn, 0, 0]],
                                     rows.at[bg], gsem.at[bg])
                else:
                    @pl.when(g + 1 < NG)
                    def _():
                        pltpu.async_copy(
                            yc.at[ibuf.at[1 - p, kn - NBLK, 0, 0]],
                            rows.at[bg], gsem.at[bg])
            return carry

        lax.fori_loop(0, NG, group, 0)
        for k in range(NBLK - GAP, NBLK):
            wait_rows(ssem.at[k % NB])
            if with_counts:
                wait_ones(csem.at[k % NB])
        plsc.subcore_barrier()
        pltpu.sync_copy(acc.at[pl.ds(r0, RPT)], out.at[c, pl.ds(r0, RPT)])
        if with_counts:
            pltpu.sync_copy(cacc.at[pl.ds(r0, RPT)],
                            outc.at[pl.ds(c * NPAD + r0, RPT)])

    return pl.kernel(
        body,
        out_type=tuple(out_types) if with_counts else out_types[0],
        mesh=_sc_mesh,
        compiler_params=_sc_params,
        scratch_types=scratch,
    )


_sc_segsum = _make_segsum()


_BM = 512


def _linear_body(x_ref, w_ref, b_ref, o_ref):
    o_ref[...] = x_ref[...] @ w_ref[...] + b_ref[...]


def _tc_linear(x, w, b):
    m, k = x.shape
    h = w.shape[1]
    return pl.pallas_call(
        _linear_body,
        grid=(m // _BM,),
        in_specs=[
            pl.BlockSpec((_BM, k), lambda i: (i, 0)),
            pl.BlockSpec((k, h), lambda i: (0, 0)),
            pl.BlockSpec((1, h), lambda i: (0, 0)),
        ],
        out_specs=pl.BlockSpec((_BM, h), lambda i: (i, 0)),
        out_shape=jax.ShapeDtypeStruct((m, h), jnp.float32),
    )(x, w, b)


def _proj_body(x_ref, w_ref, o_ref):
    o_ref[0] = x_ref[...] @ w_ref[0]


def _tc_proj(x, w):
    w2 = w.reshape(H, 2, HH).transpose(1, 0, 2)
    return pl.pallas_call(
        _proj_body,
        grid=(NPAD // _BM, 2),
        in_specs=[
            pl.BlockSpec((_BM, H), lambda i, h: (i, 0)),
            pl.BlockSpec((1, H, HH), lambda i, h: (h, 0, 0)),
        ],
        out_specs=pl.BlockSpec((1, _BM, HH), lambda i, h: (h, i, 0)),
        out_shape=jax.ShapeDtypeStruct((2, NPAD, HH), jnp.float32),
    )(x, w2)


def _tail_body(s0_ref, s1_ref, cnt_ref, b_ref, x_ref, w_ref, o_ref):
    agg = jnp.concatenate([s0_ref[0], s1_ref[0]], axis=1)
    inv = 1.0 / jnp.maximum(cnt_ref[...], 1.0)
    o_ref[...] = jnp.maximum(
        agg * inv + b_ref[...] + x_ref[...] @ w_ref[...], 0.0)


def _tc_tail(s2, cnt2d, b, x, wr):
    return pl.pallas_call(
        _tail_body,
        grid=(NPAD // _BM,),
        in_specs=[
            pl.BlockSpec((1, _BM, HH), lambda i: (0, i, 0)),
            pl.BlockSpec((1, _BM, HH), lambda i: (1, i, 0)),
            pl.BlockSpec((_BM, 1), lambda i: (i, 0)),
            pl.BlockSpec((1, H), lambda i: (0, 0)),
            pl.BlockSpec((_BM, H), lambda i: (i, 0)),
            pl.BlockSpec((H, H), lambda i: (0, 0)),
        ],
        out_specs=pl.BlockSpec((_BM, H), lambda i: (i, 0)),
        out_shape=jax.ShapeDtypeStruct((NPAD, H), jnp.float32),
    )(s2, s2, cnt2d, b, x, wr)


def _linproj_body(x_ref, w_ref, b_ref, wn_ref, o_ref, y_ref):
    h = x_ref[...] @ w_ref[...] + b_ref[...]
    o_ref[...] = h
    y_ref[0] = h @ wn_ref[0]


def _tc_linear_proj(x, w, b, wn):
    m, k = x.shape
    wn2 = wn.reshape(H, 2, HH).transpose(1, 0, 2)
    return pl.pallas_call(
        _linproj_body,
        grid=(m // _BM, 2),
        in_specs=[
            pl.BlockSpec((_BM, k), lambda i, h: (i, 0)),
            pl.BlockSpec((k, H), lambda i, h: (0, 0)),
            pl.BlockSpec((1, H), lambda i, h: (0, 0)),
            pl.BlockSpec((1, H, HH), lambda i, h: (h, 0, 0)),
        ],
        out_specs=[
            pl.BlockSpec((_BM, H), lambda i, h: (i, 0)),
            pl.BlockSpec((1, _BM, HH), lambda i, h: (h, i, 0)),
        ],
        out_shape=[
            jax.ShapeDtypeStruct((m, H), jnp.float32),
            jax.ShapeDtypeStruct((2, m, HH), jnp.float32),
        ],
    )(x, w, b, wn2)


def _tailproj_body(s0_ref, s1_ref, cnt_ref, b_ref, x_ref, w_ref, wn_ref,
                   o_ref, y_ref):
    agg = jnp.concatenate([s0_ref[0], s1_ref[0]], axis=1)
    inv = 1.0 / jnp.maximum(cnt_ref[...], 1.0)
    h = jnp.maximum(agg * inv + b_ref[...] + x_ref[...] @ w_ref[...], 0.0)
    o_ref[...] = h
    y_ref[0] = h @ wn_ref[0]


def _tc_tail_proj(s2, cnt2d, b, x, wr, wn):
    wn2 = wn.reshape(H, 2, HH).transpose(1, 0, 2)
    return pl.pallas_call(
        _tailproj_body,
        grid=(NPAD // _BM, 2),
        in_specs=[
            pl.BlockSpec((1, _BM, HH), lambda i, h: (0, i, 0)),
            pl.BlockSpec((1, _BM, HH), lambda i, h: (1, i, 0)),
            pl.BlockSpec((_BM, 1), lambda i, h: (i, 0)),
            pl.BlockSpec((1, H), lambda i, h: (0, 0)),
            pl.BlockSpec((_BM, H), lambda i, h: (i, 0)),
            pl.BlockSpec((H, H), lambda i, h: (0, 0)),
            pl.BlockSpec((1, H, HH), lambda i, h: (h, 0, 0)),
        ],
        out_specs=[
            pl.BlockSpec((_BM, H), lambda i, h: (i, 0)),
            pl.BlockSpec((1, _BM, HH), lambda i, h: (h, i, 0)),
        ],
        out_shape=[
            jax.ShapeDtypeStruct((NPAD, H), jnp.float32),
            jax.ShapeDtypeStruct((2, NPAD, HH), jnp.float32),
        ],
    )(s2, s2, cnt2d, b, x, wr, wn2)


def kernel(x_user, x_recipe, edge_u2r, edge_r2u, emb_user, W_in, b_in,
           W_ur0, Wr_ur0, b_ur0, W_ru0, Wr_ru0, b_ru0,
           W_ur1, Wr_ur1, b_ur1, W_ru1, Wr_ru1, b_ru1):
    f32 = jnp.float32

    idx_u = jnp.pad(x_user.astype(jnp.int32),
                    (0, NPAD - N)).reshape(-1, 1, GCHUNK)
    xr = jnp.pad(x_recipe, ((0, NPAD - N), (0, 16 - D_IN)))
    w_in16 = jnp.pad(W_in, ((0, 16 - D_IN), (0, 0)))

    def prep_edges(edge):
        src = jnp.pad(edge[0].astype(jnp.int32), (0, EPAD - E))
        dst = jnp.pad(edge[1].astype(jnp.int32), (0, EPAD - E),
                      constant_values=N)
        return jnp.stack([src.reshape(NS * NG, NBLK, 1, CHUNK),
                          dst.reshape(NS * NG, NBLK, 1, CHUNK)], axis=2)

    em_u2r = prep_edges(edge_u2r)
    em_r2u = prep_edges(edge_r2u)

    zeros1 = jnp.zeros((NPAD,), f32)
    zeros2 = jnp.zeros((NPAD, HH), f32)

    b2 = {k: v.reshape(1, H) for k, v in dict(
        b_in=b_in, b_ur0=b_ur0, b_ru0=b_ru0, b_ur1=b_ur1, b_ru1=b_ru1).items()}

    dsts = jnp.stack([em_u2r[:, :, 1], em_r2u[:, :, 1]])
    cnts = _sc_counts(dsts, zeros1)
    cnt_r = cnts[:NPAD].reshape(NPAD, 1)
    cnt_u = cnts[NPAD:].reshape(NPAD, 1)

    h_u = _sc_embed(emb_user, idx_u)
    h_r, y_r = _tc_linear_proj(xr, w_in16, b2["b_in"], W_ru0)
    y_u = _tc_proj(h_u, W_ur0)

    s_r = _sc_segsum(y_u, em_u2r, zeros2, zeros1)
    s_u = _sc_segsum(y_r, em_r2u, zeros2, zeros1)
    h_r1, y_r1 = _tc_tail_proj(s_r, cnt_r, b2["b_ur0"], h_r, Wr_ur0, W_ru1)
    h_u1, y_u1 = _tc_tail_proj(s_u, cnt_u, b2["b_ru0"], h_u, Wr_ru0, W_ur1)

    s_r = _sc_segsum(y_u1, em_u2r, zeros2, zeros1)
    s_u = _sc_segsum(y_r1, em_r2u, zeros2, zeros1)
    out_r = _tc_tail(s_r, cnt_r, b2["b_ur1"], h_r1, Wr_ur1)
    out_u = _tc_tail(s_u, cnt_u, b2["b_ru1"], h_u1, Wr_ru1)

    return out_u[:N], out_r[:N]


# speedup vs baseline: 1.6229x; 1.3708x over previous
import functools

import jax
import jax.numpy as jnp
from jax import lax
from jax.experimental import pallas as pl
from jax.experimental.pallas import tpu as pltpu
from jax.experimental.pallas import tpu_sc as plsc

N = 50000
E = 800000
H = 64
HH = 32
D_IN = 9

NC = 2
NS = 16
CHUNK = 128

NPAD = 50176
RPT = NPAD // NS
NCHUNK = 400
EPT = NCHUNK * CHUNK
EPAD = NS * EPT
NBLK = 8
NG = NCHUNK // NBLK
NB = 4
GAP = 2

GB = NPAD // (NC * NS)
GCHUNK = 112
GN = GB // GCHUNK

_sc_mesh = plsc.VectorSubcoreMesh(core_axis_name="c", subcore_axis_name="s")
_sc_params = pltpu.CompilerParams(use_tc_tiling_on_sc=False)


@functools.partial(
    pl.kernel,
    out_type=jax.ShapeDtypeStruct((NPAD, H), jnp.float32),
    mesh=_sc_mesh,
    compiler_params=_sc_params,
    scratch_types=[
        pltpu.VMEM((GN, 1, GCHUNK), jnp.int32),
        pltpu.VMEM((NB, GCHUNK, H), jnp.float32),
        pltpu.SemaphoreType.DMA((NB,)),
        pltpu.SemaphoreType.DMA((NB,)),
    ],
)
def _sc_embed(table, idx, out, idx_v, rows_v, gsem, osem):
    c = lax.axis_index("c")
    s = lax.axis_index("s")
    wid = s * NC + c
    base = wid * GB
    pltpu.sync_copy(idx.at[pl.ds(wid * GN, GN)], idx_v)
    for b in range(GAP):
        pltpu.async_copy(table.at[idx_v.at[b, 0]], rows_v.at[b],
                         gsem.at[b])
    for j in range(GN):
        b = j % NB
        pltpu.make_async_copy(table.at[pl.ds(0, GCHUNK)], rows_v.at[b],
                              gsem.at[b]).wait()
        pltpu.async_copy(rows_v.at[b],
                         out.at[pl.ds(base + j * GCHUNK, GCHUNK)],
                         osem.at[b])
        jn = j + GAP
        if jn < GN:
            bg = jn % NB
            if jn >= NB:
                pltpu.make_async_copy(table.at[pl.ds(0, GCHUNK)],
                                      rows_v.at[bg], osem.at[bg]).wait()
            pltpu.async_copy(table.at[idx_v.at[jn, 0]], rows_v.at[bg],
                             gsem.at[bg])
    for j in range(GN - min(GN, NB), GN):
        b = j % NB
        pltpu.make_async_copy(table.at[pl.ds(0, GCHUNK)], rows_v.at[b],
                              osem.at[b]).wait()


@functools.partial(
    pl.kernel,
    out_type=jax.ShapeDtypeStruct((2 * NPAD,), jnp.float32),
    mesh=_sc_mesh,
    compiler_params=_sc_params,
    scratch_types=[
        pltpu.VMEM((2, NBLK, 1, CHUNK), jnp.int32),
        pltpu.VMEM((CHUNK,), jnp.float32),
        pltpu.VMEM_SHARED((NPAD,), jnp.float32),
        pltpu.SemaphoreType.DMA((2,)),
        pltpu.SemaphoreType.DMA((NB,)),
    ],
)
def _sc_counts(dsts, zeros1, out, ibuf, ones_v, acc, isem, csem):
    c = lax.axis_index("c")
    s = lax.axis_index("s")
    r0 = s * RPT
    dc = dsts.at[c]
    gbase = s * NG

    def wait_ones(sem):
        pltpu.make_async_copy(zeros1.at[pl.ds(0, CHUNK)], ones_v,
                              sem).wait()

    pltpu.sync_copy(zeros1.at[pl.ds(r0, RPT)], acc.at[pl.ds(r0, RPT)])
    for i in range(CHUNK // 16):
        ones_v[pl.ds(i * 16, 16)] = jnp.ones((16,), jnp.float32)
    pltpu.sync_copy(dc.at[gbase], ibuf.at[0])
    pltpu.async_copy(dc.at[gbase + 1], ibuf.at[1], isem.at[1])
    plsc.subcore_barrier()

    def group(g, carry):
        p = lax.rem(g, 2)
        for k in range(NBLK):
            b = k % NB
            if k == 4:
                @pl.when(jnp.logical_and(g >= 1, g + 1 < NG))
                def _():
                    pltpu.async_copy(dc.at[gbase + g + 1], ibuf.at[1 - p],
                                     isem.at[1 - p])
            if k == NBLK - 1:
                @pl.when(g + 1 < NG)
                def _():
                    pltpu.make_async_copy(dc.at[gbase], ibuf.at[1 - p],
                                          isem.at[1 - p]).wait()
            if k < NB:
                @pl.when(g > 0)
                def _():
                    wait_ones(csem.at[b])
            else:
                wait_ones(csem.at[b])
            pltpu.async_copy(ones_v, acc.at[ibuf.at[p, k, 0]],
                             csem.at[b], add=True)
        return carry

    lax.fori_loop(0, NG, group, 0)
    for k in range(NBLK - NB, NBLK):
        wait_ones(csem.at[k % NB])
    plsc.subcore_barrier()
    pltpu.sync_copy(acc.at[pl.ds(r0, RPT)],
                    out.at[pl.ds(c * NPAD + r0, RPT)])


def _make_segsum():
    out_types = [jax.ShapeDtypeStruct((2, NPAD, HH), jnp.float32)]
    scratch = [
        pltpu.VMEM((2, NBLK, 2, 1, CHUNK), jnp.int32),
        pltpu.VMEM((NB, CHUNK, HH), jnp.float32),
        pltpu.VMEM_SHARED((NPAD, HH), jnp.float32),
        pltpu.SemaphoreType.DMA((2,)),
        pltpu.SemaphoreType.DMA((NB,)),
        pltpu.SemaphoreType.DMA((NB,)),
    ]
    with_counts = False

    def body(y2, em, zeros2, zeros1, *refs):
        if with_counts:
            (out, outc, ibuf, rows, acc, isem, gsem, ssem,
             ones_v, cacc, csem) = refs
        else:
            out, ibuf, rows, acc, isem, gsem, ssem = refs
        c = lax.axis_index("c")
        s = lax.axis_index("s")
        r0 = s * RPT
        yc = y2.at[c]
        gbase = s * NG

        def wait_rows(sem):
            pltpu.make_async_copy(zeros2.at[pl.ds(0, CHUNK)],
                                  rows.at[0], sem).wait()

        def wait_ones(sem):
            pltpu.make_async_copy(zeros1.at[pl.ds(0, CHUNK)], ones_v,
                                  sem).wait()

        pltpu.sync_copy(zeros2.at[pl.ds(r0, RPT)], acc.at[pl.ds(r0, RPT)])
        if with_counts:
            pltpu.sync_copy(zeros1.at[pl.ds(r0, RPT)],
                            cacc.at[pl.ds(r0, RPT)])
            for i in range(CHUNK // 16):
                ones_v[pl.ds(i * 16, 16)] = jnp.ones((16,), jnp.float32)
        pltpu.sync_copy(em.at[gbase], ibuf.at[0])
        pltpu.async_copy(em.at[gbase + 1], ibuf.at[1], isem.at[1])
        for b in range(GAP):
            pltpu.async_copy(yc.at[ibuf.at[0, b, 0, 0]], rows.at[b],
                             gsem.at[b])
        plsc.subcore_barrier()

        def group(g, carry):
            p = lax.rem(g, 2)
            for k in range(NBLK):
                b = k % NB
                if k == 2:
                    @pl.when(jnp.logical_and(g >= 1, g + 1 < NG))
                    def _():
                        pltpu.async_copy(em.at[gbase + g + 1],
                                         ibuf.at[1 - p], isem.at[1 - p])
                if k == NBLK - GAP:
                    @pl.when(g + 1 < NG)
                    def _():
                        pltpu.make_async_copy(
                            em.at[gbase], ibuf.at[1 - p],
                            isem.at[1 - p]).wait()
                wait_rows(gsem.at[b])
                pltpu.async_copy(rows.at[b], acc.at[ibuf.at[p, k, 1, 0]],
                                 ssem.at[b], add=True)
                if with_counts:
                    pltpu.async_copy(ones_v,
                                     cacc.at[ibuf.at[p, k, 1, 0]],
                                     csem.at[b], add=True)
                bg = (k + GAP) % NB

                def drain():
                    wait_rows(ssem.at[bg])
                    if with_counts:
                        wait_ones(csem.at[bg])

                if k < GAP:
                    @pl.when(g > 0)
                    def _():
                        drain()
                else:
                    drain()
                kn = k + GAP
                if kn < NBLK:
                    pltpu.async_copy(yc.at[ibuf.at[p, kn, 0, 0]],
                                     rows.at[bg], gsem.at[bg])
                else:
                    @pl.when(g + 1 < NG)
                    def _():
                        pltpu.async_copy(
                            yc.at[ibuf.at[1 - p, kn - NBLK, 0, 0]],
                            rows.at[bg], gsem.at[bg])
            return carry

        lax.fori_loop(0, NG, group, 0)
        for k in range(NBLK - GAP, NBLK):
            wait_rows(ssem.at[k % NB])
            if with_counts:
                wait_ones(csem.at[k % NB])
        plsc.subcore_barrier()
        pltpu.sync_copy(acc.at[pl.ds(r0, RPT)], out.at[c, pl.ds(r0, RPT)])
        if with_counts:
            pltpu.sync_copy(cacc.at[pl.ds(r0, RPT)],
                            outc.at[pl.ds(c * NPAD + r0, RPT)])

    return pl.kernel(
        body,
        out_type=tuple(out_types) if with_counts else out_types[0],
        mesh=_sc_mesh,
        compiler_params=_sc_params,
        scratch_types=scratch,
    )


_sc_segsum = _make_segsum()


_BM = 512


def _linear_body(x_ref, w_ref, b_ref, o_ref):
    o_ref[...] = x_ref[...] @ w_ref[...] + b_ref[...]


def _tc_linear(x, w, b):
    m, k = x.shape
    h = w.shape[1]
    return pl.pallas_call(
        _linear_body,
        grid=(m // _BM,),
        in_specs=[
            pl.BlockSpec((_BM, k), lambda i: (i, 0)),
            pl.BlockSpec((k, h), lambda i: (0, 0)),
            pl.BlockSpec((1, h), lambda i: (0, 0)),
        ],
        out_specs=pl.BlockSpec((_BM, h), lambda i: (i, 0)),
        out_shape=jax.ShapeDtypeStruct((m, h), jnp.float32),
    )(x, w, b)


def _proj_body(x_ref, w_ref, o_ref):
    o_ref[0] = x_ref[...] @ w_ref[0]


def _tc_proj(x, w):
    w2 = w.reshape(H, 2, HH).transpose(1, 0, 2)
    return pl.pallas_call(
        _proj_body,
        grid=(NPAD // _BM, 2),
        in_specs=[
            pl.BlockSpec((_BM, H), lambda i, h: (i, 0)),
            pl.BlockSpec((1, H, HH), lambda i, h: (h, 0, 0)),
        ],
        out_specs=pl.BlockSpec((1, _BM, HH), lambda i, h: (h, i, 0)),
        out_shape=jax.ShapeDtypeStruct((2, NPAD, HH), jnp.float32),
    )(x, w2)


def _tail_body(s0_ref, s1_ref, cnt_ref, b_ref, x_ref, w_ref, o_ref):
    agg = jnp.concatenate([s0_ref[0], s1_ref[0]], axis=1)
    inv = 1.0 / jnp.maximum(cnt_ref[...], 1.0)
    o_ref[...] = jnp.maximum(
        agg * inv + b_ref[...] + x_ref[...] @ w_ref[...], 0.0)


def _tc_tail(s2, cnt2d, b, x, wr):
    return pl.pallas_call(
        _tail_body,
        grid=(NPAD // _BM,),
        in_specs=[
            pl.BlockSpec((1, _BM, HH), lambda i: (0, i, 0)),
            pl.BlockSpec((1, _BM, HH), lambda i: (1, i, 0)),
            pl.BlockSpec((_BM, 1), lambda i: (i, 0)),
            pl.BlockSpec((1, H), lambda i: (0, 0)),
            pl.BlockSpec((_BM, H), lambda i: (i, 0)),
            pl.BlockSpec((H, H), lambda i: (0, 0)),
        ],
        out_specs=pl.BlockSpec((_BM, H), lambda i: (i, 0)),
        out_shape=jax.ShapeDtypeStruct((NPAD, H), jnp.float32),
    )(s2, s2, cnt2d, b, x, wr)


def _linproj_body(x_ref, w_ref, b_ref, wn_ref, o_ref, y_ref):
    h = x_ref[...] @ w_ref[...] + b_ref[...]
    o_ref[...] = h
    y_ref[0] = h @ wn_ref[0]


def _tc_linear_proj(x, w, b, wn):
    m, k = x.shape
    wn2 = wn.reshape(H, 2, HH).transpose(1, 0, 2)
    return pl.pallas_call(
        _linproj_body,
        grid=(m // _BM, 2),
        in_specs=[
            pl.BlockSpec((_BM, k), lambda i, h: (i, 0)),
            pl.BlockSpec((k, H), lambda i, h: (0, 0)),
            pl.BlockSpec((1, H), lambda i, h: (0, 0)),
            pl.BlockSpec((1, H, HH), lambda i, h: (h, 0, 0)),
        ],
        out_specs=[
            pl.BlockSpec((_BM, H), lambda i, h: (i, 0)),
            pl.BlockSpec((1, _BM, HH), lambda i, h: (h, i, 0)),
        ],
        out_shape=[
            jax.ShapeDtypeStruct((m, H), jnp.float32),
            jax.ShapeDtypeStruct((2, m, HH), jnp.float32),
        ],
    )(x, w, b, wn2)


def _tailproj_body(s0_ref, s1_ref, cnt_ref, b_ref, x_ref, w_ref, wn_ref,
                   o_ref, y_ref):
    agg = jnp.concatenate([s0_ref[0], s1_ref[0]], axis=1)
    inv = 1.0 / jnp.maximum(cnt_ref[...], 1.0)
    h = jnp.maximum(agg * inv + b_ref[...] + x_ref[...] @ w_ref[...], 0.0)
    o_ref[...] = h
    y_ref[0] = h @ wn_ref[0]


def _tc_tail_proj(s2, cnt2d, b, x, wr, wn):
    wn2 = wn.reshape(H, 2, HH).transpose(1, 0, 2)
    return pl.pallas_call(
        _tailproj_body,
        grid=(NPAD // _BM, 2),
        in_specs=[
            pl.BlockSpec((1, _BM, HH), lambda i, h: (0, i, 0)),
            pl.BlockSpec((1, _BM, HH), lambda i, h: (1, i, 0)),
            pl.BlockSpec((_BM, 1), lambda i, h: (i, 0)),
            pl.BlockSpec((1, H), lambda i, h: (0, 0)),
            pl.BlockSpec((_BM, H), lambda i, h: (i, 0)),
            pl.BlockSpec((H, H), lambda i, h: (0, 0)),
            pl.BlockSpec((1, H, HH), lambda i, h: (h, 0, 0)),
        ],
        out_specs=[
            pl.BlockSpec((_BM, H), lambda i, h: (i, 0)),
            pl.BlockSpec((1, _BM, HH), lambda i, h: (h, i, 0)),
        ],
        out_shape=[
            jax.ShapeDtypeStruct((NPAD, H), jnp.float32),
            jax.ShapeDtypeStruct((2, NPAD, HH), jnp.float32),
        ],
    )(s2, s2, cnt2d, b, x, wr, wn2)


def kernel(x_user, x_recipe, edge_u2r, edge_r2u, emb_user, W_in, b_in,
           W_ur0, Wr_ur0, b_ur0, W_ru0, Wr_ru0, b_ru0,
           W_ur1, Wr_ur1, b_ur1, W_ru1, Wr_ru1, b_ru1):
    f32 = jnp.float32

    idx_u = jnp.pad(x_user.astype(jnp.int32),
                    (0, NPAD - N)).reshape(-1, 1, GCHUNK)
    xr = jnp.pad(x_recipe, ((0, NPAD - N), (0, 16 - D_IN)))
    w_in16 = jnp.pad(W_in, ((0, 16 - D_IN), (0, 0)))

    def prep_edges(edge):
        src = jnp.pad(edge[0].astype(jnp.int32), (0, EPAD - E))
        dst = jnp.pad(edge[1].astype(jnp.int32), (0, EPAD - E),
                      constant_values=N)
        return jnp.stack([src.reshape(NS * NG, NBLK, 1, CHUNK),
                          dst.reshape(NS * NG, NBLK, 1, CHUNK)], axis=2)

    em_u2r = prep_edges(edge_u2r)
    em_r2u = prep_edges(edge_r2u)

    zeros1 = jnp.zeros((NPAD,), f32)
    zeros2 = jnp.zeros((NPAD, HH), f32)

    b2 = {k: v.reshape(1, H) for k, v in dict(
        b_in=b_in, b_ur0=b_ur0, b_ru0=b_ru0, b_ur1=b_ur1, b_ru1=b_ru1).items()}

    dsts = jnp.stack([em_u2r[:, :, 1], em_r2u[:, :, 1]])
    cnts = _sc_counts(dsts, zeros1)
    cnt_r = cnts[:NPAD].reshape(NPAD, 1)
    cnt_u = cnts[NPAD:].reshape(NPAD, 1)

    h_u = _sc_embed(emb_user, idx_u)
    h_r, y_r = _tc_linear_proj(xr, w_in16, b2["b_in"], W_ru0)
    y_u = _tc_proj(h_u, W_ur0)

    s_r = _sc_segsum(y_u, em_u2r, zeros2, zeros1)
    s_u = _sc_segsum(y_r, em_r2u, zeros2, zeros1)
    h_r1, y_r1 = _tc_tail_proj(s_r, cnt_r, b2["b_ur0"], h_r, Wr_ur0, W_ru1)
    h_u1, y_u1 = _tc_tail_proj(s_u, cnt_u, b2["b_ru0"], h_u, Wr_ru0, W_ur1)

    s_r = _sc_segsum(y_u1, em_u2r, zeros2, zeros1)
    s_u = _sc_segsum(y_r1, em_r2u, zeros2, zeros1)
    out_r = _tc_tail(s_r, cnt_r, b2["b_ur1"], h_r1, Wr_ur1)
    out_u = _tc_tail(s_u, cnt_u, b2["b_ru1"], h_u1, Wr_ru1)

    return out_u[:N], out_r[:N]


# speedup vs baseline: 2.3859x; 1.4701x over previous
import functools

import jax
import jax.numpy as jnp
from jax import lax
from jax.experimental import pallas as pl
from jax.experimental.pallas import tpu as pltpu
from jax.experimental.pallas import tpu_sc as plsc

N = 50000
E = 800000
H = 64
HH = 32
D_IN = 9

NC = 2
NS = 16
CHUNK = 128

NPAD = 50176
RPT = NPAD // NS
NCHUNK = 400
EPT = NCHUNK * CHUNK
EPAD = NS * EPT
NBLK = 8
NG = NCHUNK // NBLK
NB = 4
GAP = 2

GB = NPAD // (NC * NS)
GCHUNK = 112
GN = GB // GCHUNK

_sc_mesh = plsc.VectorSubcoreMesh(core_axis_name="c", subcore_axis_name="s")
_sc_params = pltpu.CompilerParams(use_tc_tiling_on_sc=False)


@functools.partial(
    pl.kernel,
    out_type=jax.ShapeDtypeStruct((NPAD, H), jnp.float32),
    mesh=_sc_mesh,
    compiler_params=_sc_params,
    scratch_types=[
        pltpu.VMEM((GN, 1, GCHUNK), jnp.int32),
        pltpu.VMEM((NB, GCHUNK, H), jnp.float32),
        pltpu.SemaphoreType.DMA((NB,)),
        pltpu.SemaphoreType.DMA((NB,)),
    ],
)
def _sc_embed(table, idx, out, idx_v, rows_v, gsem, osem):
    c = lax.axis_index("c")
    s = lax.axis_index("s")
    wid = s * NC + c
    base = wid * GB
    pltpu.sync_copy(idx.at[pl.ds(wid * GN, GN)], idx_v)
    for b in range(GAP):
        pltpu.async_copy(table.at[idx_v.at[b, 0]], rows_v.at[b],
                         gsem.at[b])
    for j in range(GN):
        b = j % NB
        pltpu.make_async_copy(table.at[pl.ds(0, GCHUNK)], rows_v.at[b],
                              gsem.at[b]).wait()
        pltpu.async_copy(rows_v.at[b],
                         out.at[pl.ds(base + j * GCHUNK, GCHUNK)],
                         osem.at[b])
        jn = j + GAP
        if jn < GN:
            bg = jn % NB
            if jn >= NB:
                pltpu.make_async_copy(table.at[pl.ds(0, GCHUNK)],
                                      rows_v.at[bg], osem.at[bg]).wait()
            pltpu.async_copy(table.at[idx_v.at[jn, 0]], rows_v.at[bg],
                             gsem.at[bg])
    for j in range(GN - min(GN, NB), GN):
        b = j % NB
        pltpu.make_async_copy(table.at[pl.ds(0, GCHUNK)], rows_v.at[b],
                              osem.at[b]).wait()


@functools.partial(
    pl.kernel,
    out_type=jax.ShapeDtypeStruct((2 * NPAD,), jnp.float32),
    mesh=_sc_mesh,
    compiler_params=_sc_params,
    scratch_types=[
        pltpu.VMEM((2, NBLK, 1, CHUNK), jnp.int32),
        pltpu.VMEM((CHUNK,), jnp.float32),
        pltpu.VMEM_SHARED((NPAD,), jnp.float32),
        pltpu.SemaphoreType.DMA((2,)),
        pltpu.SemaphoreType.DMA((NB,)),
    ],
)
def _sc_counts(dsts, zeros1, out, ibuf, ones_v, acc, isem, csem):
    c = lax.axis_index("c")
    s = lax.axis_index("s")
    r0 = s * RPT
    dc = dsts.at[c]
    gbase = s * NG

    def wait_ones(sem):
        pltpu.make_async_copy(zeros1.at[pl.ds(0, CHUNK)], ones_v,
                              sem).wait()

    pltpu.sync_copy(zeros1.at[pl.ds(r0, RPT)], acc.at[pl.ds(r0, RPT)])
    for i in range(CHUNK // 16):
        ones_v[pl.ds(i * 16, 16)] = jnp.ones((16,), jnp.float32)
    pltpu.sync_copy(dc.at[gbase], ibuf.at[0])
    pltpu.async_copy(dc.at[gbase + 1], ibuf.at[1], isem.at[1])
    plsc.subcore_barrier()

    def group(g, carry):
        p = lax.rem(g, 2)
        for k in range(NBLK):
            b = k % NB
            if k == 4:
                @pl.when(jnp.logical_and(g >= 1, g + 1 < NG))
                def _():
                    pltpu.async_copy(dc.at[gbase + g + 1], ibuf.at[1 - p],
                                     isem.at[1 - p])
            if k == NBLK - 1:
                @pl.when(g + 1 < NG)
                def _():
                    pltpu.make_async_copy(dc.at[gbase], ibuf.at[1 - p],
                                          isem.at[1 - p]).wait()
            if k < NB:
                @pl.when(g > 0)
                def _():
                    wait_ones(csem.at[b])
            else:
                wait_ones(csem.at[b])
            pltpu.async_copy(ones_v, acc.at[ibuf.at[p, k, 0]],
                             csem.at[b], add=True)
        return carry

    lax.fori_loop(0, NG, group, 0)
    for k in range(NBLK - NB, NBLK):
        wait_ones(csem.at[k % NB])
    plsc.subcore_barrier()
    pltpu.sync_copy(acc.at[pl.ds(r0, RPT)],
                    out.at[pl.ds(c * NPAD + r0, RPT)])


def _make_segsum():
    out_types = [jax.ShapeDtypeStruct((2, NPAD, HH), jnp.float32)]
    scratch = [
        pltpu.VMEM((2, NBLK, 2, 1, CHUNK), jnp.int32),
        pltpu.VMEM((NB, CHUNK, HH), jnp.float32),
        pltpu.VMEM_SHARED((NPAD, HH), jnp.float32),
        pltpu.SemaphoreType.DMA((2,)),
        pltpu.SemaphoreType.DMA((NB,)),
        pltpu.SemaphoreType.DMA((NB,)),
    ]
    with_counts = False

    def body(y2, em, zeros2, zeros1, *refs):
        if with_counts:
            (out, outc, ibuf, rows, acc, isem, gsem, ssem,
             ones_v, cacc, csem) = refs
        else:
            out, ibuf, rows, acc, isem, gsem, ssem = refs
        c = lax.axis_index("c")
        s = lax.axis_index("s")
        r0 = s * RPT
        yc = y2.at[c]
        gbase = s * NG

        def wait_rows(sem):
            pltpu.make_async_copy(zeros2.at[pl.ds(0, CHUNK)],
                                  rows.at[0], sem).wait()

        def wait_ones(sem):
            pltpu.make_async_copy(zeros1.at[pl.ds(0, CHUNK)], ones_v,
                                  sem).wait()

        pltpu.sync_copy(zeros2.at[pl.ds(r0, RPT)], acc.at[pl.ds(r0, RPT)])
        if with_counts:
            pltpu.sync_copy(zeros1.at[pl.ds(r0, RPT)],
                            cacc.at[pl.ds(r0, RPT)])
            for i in range(CHUNK // 16):
                ones_v[pl.ds(i * 16, 16)] = jnp.ones((16,), jnp.float32)
        pltpu.sync_copy(em.at[gbase], ibuf.at[0])
        pltpu.async_copy(em.at[gbase + 1], ibuf.at[1], isem.at[1])
        for b in range(GAP):
            pltpu.async_copy(yc.at[ibuf.at[0, b, 0, 0]], rows.at[b],
                             gsem.at[b])
        plsc.subcore_barrier()

        def group(g, carry):
            p = lax.rem(g, 2)
            for k in range(NBLK):
                b = k % NB
                if k == 2:
                    @pl.when(jnp.logical_and(g >= 1, g + 1 < NG))
                    def _():
                        pltpu.async_copy(em.at[gbase + g + 1],
                                         ibuf.at[1 - p], isem.at[1 - p])
                if k == NBLK - GAP:
                    @pl.when(g + 1 < NG)
                    def _():
                        pltpu.make_async_copy(
                            em.at[gbase], ibuf.at[1 - p],
                            isem.at[1 - p]).wait()
                wait_rows(gsem.at[b])
                pltpu.async_copy(rows.at[b], acc.at[ibuf.at[p, k, 1, 0]],
                                 ssem.at[b], add=True)
                if with_counts:
                    pltpu.async_copy(ones_v,
                                     cacc.at[ibuf.at[p, k, 1, 0]],
                                     csem.at[b], add=True)
                bg = (k + GAP) % NB

                def drain():
                    wait_rows(ssem.at[bg])
                    if with_counts:
                        wait_ones(csem.at[bg])

                if k < GAP:
                    @pl.when(g > 0)
                    def _():
                        drain()
                else:
                    drain()
                kn = k + GAP
                if kn < NBLK:
                    pltpu.async_copy(yc.at[ibuf.at[p, kn, 0, 0]],
                                     rows.at[bg], gsem.at[bg])
                else:
                    @pl.when(g + 1 < NG)
                    def _():
                        pltpu.async_copy(
                            yc.at[ibuf.at[1 - p, kn - NBLK, 0, 0]],
                            rows.at[bg], gsem.at[bg])
            return carry

        lax.fori_loop(0, NG, group, 0)
        for k in range(NBLK - GAP, NBLK):
            wait_rows(ssem.at[k % NB])
            if with_counts:
                wait_ones(csem.at[k % NB])
        plsc.subcore_barrier()
        pltpu.sync_copy(acc.at[pl.ds(r0, RPT)], out.at[c, pl.ds(r0, RPT)])
        if with_counts:
            pltpu.sync_copy(cacc.at[pl.ds(r0, RPT)],
                            outc.at[pl.ds(c * NPAD + r0, RPT)])

    return pl.kernel(
        body,
        out_type=tuple(out_types) if with_counts else out_types[0],
        mesh=_sc_mesh,
        compiler_params=_sc_params,
        scratch_types=scratch,
    )


_sc_segsum = _make_segsum()


_BM = 512


def _linear_body(x_ref, w_ref, b_ref, o_ref):
    o_ref[...] = x_ref[...] @ w_ref[...] + b_ref[...]


def _tc_linear(x, w, b):
    m, k = x.shape
    h = w.shape[1]
    return pl.pallas_call(
        _linear_body,
        grid=(m // _BM,),
        in_specs=[
            pl.BlockSpec((_BM, k), lambda i: (i, 0)),
            pl.BlockSpec((k, h), lambda i: (0, 0)),
            pl.BlockSpec((1, h), lambda i: (0, 0)),
        ],
        out_specs=pl.BlockSpec((_BM, h), lambda i: (i, 0)),
        out_shape=jax.ShapeDtypeStruct((m, h), jnp.float32),
    )(x, w, b)


def _proj_body(x_ref, w_ref, o_ref):
    o_ref[0] = x_ref[...] @ w_ref[0]


def _tc_proj(x, w):
    w2 = w.reshape(H, 2, HH).transpose(1, 0, 2)
    return pl.pallas_call(
        _proj_body,
        grid=(NPAD // _BM, 2),
        in_specs=[
            pl.BlockSpec((_BM, H), lambda i, h: (i, 0)),
            pl.BlockSpec((1, H, HH), lambda i, h: (h, 0, 0)),
        ],
        out_specs=pl.BlockSpec((1, _BM, HH), lambda i, h: (h, i, 0)),
        out_shape=jax.ShapeDtypeStruct((2, NPAD, HH), jnp.float32),
    )(x, w2)


def _tail_body(s0_ref, s1_ref, cnt_ref, b_ref, x_ref, w_ref, o_ref):
    agg = jnp.concatenate([s0_ref[0], s1_ref[0]], axis=1)
    inv = 1.0 / jnp.maximum(cnt_ref[...], 1.0)
    o_ref[...] = jnp.maximum(
        agg * inv + b_ref[...] + x_ref[...] @ w_ref[...], 0.0)


def _tc_tail(s2, cnt2d, b, x, wr):
    return pl.pallas_call(
        _tail_body,
        grid=(NPAD // _BM,),
        in_specs=[
            pl.BlockSpec((1, _BM, HH), lambda i: (0, i, 0)),
            pl.BlockSpec((1, _BM, HH), lambda i: (1, i, 0)),
            pl.BlockSpec((_BM, 1), lambda i: (i, 0)),
            pl.BlockSpec((1, H), lambda i: (0, 0)),
            pl.BlockSpec((_BM, H), lambda i: (i, 0)),
            pl.BlockSpec((H, H), lambda i: (0, 0)),
        ],
        out_specs=pl.BlockSpec((_BM, H), lambda i: (i, 0)),
        out_shape=jax.ShapeDtypeStruct((NPAD, H), jnp.float32),
    )(s2, s2, cnt2d, b, x, wr)


def _linproj_body(x_ref, w_ref, b_ref, wn_ref, o_ref, y_ref):
    h = x_ref[...] @ w_ref[...] + b_ref[...]
    o_ref[...] = h
    y_ref[0] = h @ wn_ref[0]


def _tc_linear_proj(x, w, b, wn):
    m, k = x.shape
    wn2 = wn.reshape(H, 2, HH).transpose(1, 0, 2)
    return pl.pallas_call(
        _linproj_body,
        grid=(m // _BM, 2),
        in_specs=[
            pl.BlockSpec((_BM, k), lambda i, h: (i, 0)),
            pl.BlockSpec((k, H), lambda i, h: (0, 0)),
            pl.BlockSpec((1, H), lambda i, h: (0, 0)),
            pl.BlockSpec((1, H, HH), lambda i, h: (h, 0, 0)),
        ],
        out_specs=[
            pl.BlockSpec((_BM, H), lambda i, h: (i, 0)),
            pl.BlockSpec((1, _BM, HH), lambda i, h: (h, i, 0)),
        ],
        out_shape=[
            jax.ShapeDtypeStruct((m, H), jnp.float32),
            jax.ShapeDtypeStruct((2, m, HH), jnp.float32),
        ],
    )(x, w, b, wn2)


def _tailproj_body(s0_ref, s1_ref, cnt_ref, b_ref, x_ref, w_ref, wn_ref,
                   o_ref, y_ref):
    agg = jnp.concatenate([s0_ref[0], s1_ref[0]], axis=1)
    inv = 1.0 / jnp.maximum(cnt_ref[...], 1.0)
    h = jnp.maximum(agg * inv + b_ref[...] + x_ref[...] @ w_ref[...], 0.0)
    o_ref[...] = h
    y_ref[0] = h @ wn_ref[0]


def _tc_tail_proj(s2, cnt2d, b, x, wr, wn):
    wn2 = wn.reshape(H, 2, HH).transpose(1, 0, 2)
    return pl.pallas_call(
        _tailproj_body,
        grid=(NPAD // _BM, 2),
        in_specs=[
            pl.BlockSpec((1, _BM, HH), lambda i, h: (0, i, 0)),
            pl.BlockSpec((1, _BM, HH), lambda i, h: (1, i, 0)),
            pl.BlockSpec((_BM, 1), lambda i, h: (i, 0)),
            pl.BlockSpec((1, H), lambda i, h: (0, 0)),
            pl.BlockSpec((_BM, H), lambda i, h: (i, 0)),
            pl.BlockSpec((H, H), lambda i, h: (0, 0)),
            pl.BlockSpec((1, H, HH), lambda i, h: (h, 0, 0)),
        ],
        out_specs=[
            pl.BlockSpec((_BM, H), lambda i, h: (i, 0)),
            pl.BlockSpec((1, _BM, HH), lambda i, h: (h, i, 0)),
        ],
        out_shape=[
            jax.ShapeDtypeStruct((NPAD, H), jnp.float32),
            jax.ShapeDtypeStruct((2, NPAD, HH), jnp.float32),
        ],
    )(s2, s2, cnt2d, b, x, wr, wn2)


def kernel(x_user, x_recipe, edge_u2r, edge_r2u, emb_user, W_in, b_in,
           W_ur0, Wr_ur0, b_ur0, W_ru0, Wr_ru0, b_ru0,
           W_ur1, Wr_ur1, b_ur1, W_ru1, Wr_ru1, b_ru1):
    f32 = jnp.float32

    idx_u = jnp.pad(x_user.astype(jnp.int32),
                    (0, NPAD - N)).reshape(-1, 1, GCHUNK)
    xr = jnp.pad(x_recipe, ((0, NPAD - N), (0, 16 - D_IN)))
    w_in16 = jnp.pad(W_in, ((0, 16 - D_IN), (0, 0)))

    pad_ar = jnp.arange(EPAD - E, dtype=jnp.int32)

    def prep_edges(edge):
        src = jnp.concatenate([edge[0].astype(jnp.int32), pad_ar % N])
        dst = jnp.concatenate([edge[1].astype(jnp.int32),
                               N + pad_ar % (NPAD - N)])
        return jnp.stack([src.reshape(NS * NG, NBLK, 1, CHUNK),
                          dst.reshape(NS * NG, NBLK, 1, CHUNK)], axis=2)

    em_u2r = prep_edges(edge_u2r)
    em_r2u = prep_edges(edge_r2u)

    zeros1 = jnp.zeros((NPAD,), f32)
    zeros2 = jnp.zeros((NPAD, HH), f32)

    b2 = {k: v.reshape(1, H) for k, v in dict(
        b_in=b_in, b_ur0=b_ur0, b_ru0=b_ru0, b_ur1=b_ur1, b_ru1=b_ru1).items()}

    dsts = jnp.stack([em_u2r[:, :, 1], em_r2u[:, :, 1]])
    cnts = _sc_counts(dsts, zeros1)
    cnt_r = cnts[:NPAD].reshape(NPAD, 1)
    cnt_u = cnts[NPAD:].reshape(NPAD, 1)

    h_u = _sc_embed(emb_user, idx_u)
    h_r, y_r = _tc_linear_proj(xr, w_in16, b2["b_in"], W_ru0)
    y_u = _tc_proj(h_u, W_ur0)

    s_r = _sc_segsum(y_u, em_u2r, zeros2, zeros1)
    s_u = _sc_segsum(y_r, em_r2u, zeros2, zeros1)
    h_r1, y_r1 = _tc_tail_proj(s_r, cnt_r, b2["b_ur0"], h_r, Wr_ur0, W_ru1)
    h_u1, y_u1 = _tc_tail_proj(s_u, cnt_u, b2["b_ru0"], h_u, Wr_ru0, W_ur1)

    s_r = _sc_segsum(y_u1, em_u2r, zeros2, zeros1)
    s_u = _sc_segsum(y_r1, em_r2u, zeros2, zeros1)
    out_r = _tc_tail(s_r, cnt_r, b2["b_ur1"], h_r1, Wr_ur1)
    out_u = _tc_tail(s_u, cnt_u, b2["b_ru1"], h_u1, Wr_ru1)

    return out_u[:N], out_r[:N]
